# baseline (device time: 371831 ns/iter reference)
import jax
import jax.numpy as jnp
from jax import lax
from jax.experimental import pallas as pl
from jax.experimental.pallas import tpu as pltpu

N_DEV = 16
M = 4096
N_OUT = 2048
CHUNK = M // N_DEV
HALF = N_OUT // 2
COMM_DTYPE = jnp.bfloat16

MESH = pl.DeviceIdType.MESH
N_STEPS = 2 * (N_DEV - 1)


def _body(x_ref, w_ref, sx_ref, sw_ref, out_ref, p_ref,
          cw_ref, ccw_ref, cw_send_sems, cw_recv_sems,
          ccw_send_sems, ccw_recv_sems, cw_credit, ccw_credit):
    my = lax.axis_index("i")
    left = (my - 1) % N_DEV
    right = (my + 1) % N_DEV

    barrier = pltpu.get_barrier_semaphore()
    for nbr in (left, right):
        pl.semaphore_signal(barrier, inc=1, device_id=(nbr,),
                            device_id_type=MESH)
    pl.semaphore_wait(barrier, 2)

    scale = sx_ref[0] * sw_ref[0]
    p_ref[:, :] = lax.dot_general(
        x_ref[:, :].astype(jnp.bfloat16),
        w_ref[:, :].astype(jnp.bfloat16),
        (((1,), (0,)), ((), ())),
        preferred_element_type=jnp.float32,
    ).astype(COMM_DTYPE)

    def cw_rows(c):
        return (pl.ds(c * CHUNK, CHUNK), pl.ds(0, HALF))

    def ccw_rows(c):
        return (pl.ds(c * CHUNK, CHUNK), pl.ds(HALF, HALF))

    def mk_cw(g):
        return pltpu.make_async_remote_copy(
            src_ref=cw_ref.at[g % 2],
            dst_ref=cw_ref.at[(g + 1) % 2],
            send_sem=cw_send_sems.at[g % 2],
            recv_sem=cw_recv_sems.at[(g + 1) % 2],
            device_id=(right,),
            device_id_type=MESH,
        )

    def mk_ccw(g):
        return pltpu.make_async_remote_copy(
            src_ref=ccw_ref.at[g % 2],
            dst_ref=ccw_ref.at[(g + 1) % 2],
            send_sem=ccw_send_sems.at[g % 2],
            recv_sem=ccw_recv_sems.at[(g + 1) % 2],
            device_id=(left,),
            device_id_type=MESH,
        )

    cw_ref[0, :, :] = p_ref[cw_rows(my)]
    ccw_ref[0, :, :] = p_ref[ccw_rows(my)]
    mk_cw(0).start()
    mk_ccw(0).start()

    for g in range(N_STEPS):
        slot_r = (g + 1) % 2
        cw = mk_cw(g)
        ccw = mk_ccw(g)
        is_rs = g < N_DEV - 1
        finalize = g == N_DEV - 2
        if is_rs:
            c_cw = (my - g - 1) % N_DEV
            c_ccw = (my + g + 1) % N_DEV
            cw.wait_recv()
            acc_cw = (cw_ref[slot_r, :, :].astype(jnp.float32)
                      + p_ref[cw_rows(c_cw)].astype(jnp.float32))
            cw_ref[slot_r, :, :] = acc_cw.astype(COMM_DTYPE)
            ccw.wait_recv()
            acc_ccw = (ccw_ref[slot_r, :, :].astype(jnp.float32)
                       + p_ref[ccw_rows(c_ccw)].astype(jnp.float32))
            ccw_ref[slot_r, :, :] = acc_ccw.astype(COMM_DTYPE)
        else:
            t = g - (N_DEV - 1)
            c_cw = (my - t) % N_DEV
            c_ccw = (my + t) % N_DEV
            cw.wait_recv()
            ccw.wait_recv()
        cw.wait_send()
        ccw.wait_send()
        if g < N_STEPS - 1:
            pl.semaphore_signal(cw_credit, inc=1, device_id=(left,),
                                device_id_type=MESH)
            pl.semaphore_signal(ccw_credit, inc=1, device_id=(right,),
                                device_id_type=MESH)
            pl.semaphore_wait(cw_credit, 1)
            pl.semaphore_wait(ccw_credit, 1)
            mk_cw(g + 1).start()
            mk_ccw(g + 1).start()
        if finalize:
            out_ref[cw_rows(c_cw)] = acc_cw * scale
            out_ref[ccw_rows(c_ccw)] = acc_ccw * scale
        if not is_rs:
            out_ref[cw_rows(c_cw)] = (
                cw_ref[slot_r, :, :].astype(jnp.float32) * scale)
            out_ref[ccw_rows(c_ccw)] = (
                ccw_ref[slot_r, :, :].astype(jnp.float32) * scale)


def kernel(x, w_mat, scale_x, scale_w):
    return pl.pallas_call(
        _body,
        out_shape=jax.ShapeDtypeStruct((M, N_OUT), jnp.float32),
        in_specs=[
            pl.BlockSpec(memory_space=pltpu.VMEM),
            pl.BlockSpec(memory_space=pltpu.VMEM),
            pl.BlockSpec(memory_space=pltpu.SMEM),
            pl.BlockSpec(memory_space=pltpu.SMEM),
        ],
        out_specs=pl.BlockSpec(memory_space=pltpu.VMEM),
        scratch_shapes=[
            pltpu.VMEM((M, N_OUT), COMM_DTYPE),
            pltpu.VMEM((2, CHUNK, HALF), COMM_DTYPE),
            pltpu.VMEM((2, CHUNK, HALF), COMM_DTYPE),
            pltpu.SemaphoreType.DMA((2,)),
            pltpu.SemaphoreType.DMA((2,)),
            pltpu.SemaphoreType.DMA((2,)),
            pltpu.SemaphoreType.DMA((2,)),
            pltpu.SemaphoreType.REGULAR,
            pltpu.SemaphoreType.REGULAR,
        ],
        compiler_params=pltpu.CompilerParams(
            collective_id=0,
            vmem_limit_bytes=100 * 1024 * 1024,
        ),
    )(x, w_mat, scale_x, scale_w)


# device time: 295822 ns/iter; 1.2569x vs baseline; 1.2569x over previous
import jax
import jax.numpy as jnp
from jax import lax
from jax.experimental import pallas as pl
from jax.experimental.pallas import tpu as pltpu

N_DEV = 16
M = 4096
N_OUT = 2048
CHUNK = M // N_DEV
HALF = N_OUT // 2
COMM_DTYPE = jnp.bfloat16

MESH = pl.DeviceIdType.MESH
N_STEPS = 2 * (N_DEV - 1)

RING = (0, 1, 5, 9, 13, 14, 10, 6, 2, 3, 7, 11, 15, 12, 8, 4)


def _body(x_ref, w_ref, sx_ref, sw_ref, meta_ref, out_ref, p_ref,
          cw_ref, ccw_ref, cw_send_sems, cw_recv_sems,
          ccw_send_sems, ccw_recv_sems, cw_credit, ccw_credit):
    pos = meta_ref[0]
    left = meta_ref[1]
    right = meta_ref[2]

    barrier = pltpu.get_barrier_semaphore()
    for nbr in (left, right):
        pl.semaphore_signal(barrier, inc=1, device_id=(nbr,),
                            device_id_type=MESH)
    pl.semaphore_wait(barrier, 2)

    scale = sx_ref[0] * sw_ref[0]
    p_ref[:, :] = lax.dot_general(
        x_ref[:, :].astype(jnp.bfloat16),
        w_ref[:, :].astype(jnp.bfloat16),
        (((1,), (0,)), ((), ())),
        preferred_element_type=jnp.float32,
    ).astype(COMM_DTYPE)

    def cw_rows(c):
        return (pl.ds(c * CHUNK, CHUNK), pl.ds(0, HALF))

    def ccw_rows(c):
        return (pl.ds(c * CHUNK, CHUNK), pl.ds(HALF, HALF))

    def mk_cw(g):
        return pltpu.make_async_remote_copy(
            src_ref=cw_ref.at[g % 2],
            dst_ref=cw_ref.at[(g + 1) % 2],
            send_sem=cw_send_sems.at[g % 2],
            recv_sem=cw_recv_sems.at[(g + 1) % 2],
            device_id=(right,),
            device_id_type=MESH,
        )

    def mk_ccw(g):
        return pltpu.make_async_remote_copy(
            src_ref=ccw_ref.at[g % 2],
            dst_ref=ccw_ref.at[(g + 1) % 2],
            send_sem=ccw_send_sems.at[g % 2],
            recv_sem=ccw_recv_sems.at[(g + 1) % 2],
            device_id=(left,),
            device_id_type=MESH,
        )

    cw_ref[0, :, :] = p_ref[cw_rows(pos)]
    ccw_ref[0, :, :] = p_ref[ccw_rows(pos)]
    mk_cw(0).start()
    mk_ccw(0).start()

    for g in range(N_STEPS):
        slot_r = (g + 1) % 2
        cw = mk_cw(g)
        ccw = mk_ccw(g)
        is_rs = g < N_DEV - 1
        finalize = g == N_DEV - 2
        if is_rs:
            c_cw = (pos - g - 1) % N_DEV
            c_ccw = (pos + g + 1) % N_DEV
            cw.wait_recv()
            acc_cw = (cw_ref[slot_r, :, :].astype(jnp.float32)
                      + p_ref[cw_rows(c_cw)].astype(jnp.float32))
            cw_ref[slot_r, :, :] = acc_cw.astype(COMM_DTYPE)
            ccw.wait_recv()
            acc_ccw = (ccw_ref[slot_r, :, :].astype(jnp.float32)
                       + p_ref[ccw_rows(c_ccw)].astype(jnp.float32))
            ccw_ref[slot_r, :, :] = acc_ccw.astype(COMM_DTYPE)
        else:
            t = g - (N_DEV - 1)
            c_cw = (pos - t) % N_DEV
            c_ccw = (pos + t) % N_DEV
            cw.wait_recv()
            ccw.wait_recv()
        cw.wait_send()
        ccw.wait_send()
        if g < N_STEPS - 1:
            pl.semaphore_signal(cw_credit, inc=1, device_id=(left,),
                                device_id_type=MESH)
            pl.semaphore_signal(ccw_credit, inc=1, device_id=(right,),
                                device_id_type=MESH)
            pl.semaphore_wait(cw_credit, 1)
            pl.semaphore_wait(ccw_credit, 1)
            mk_cw(g + 1).start()
            mk_ccw(g + 1).start()
        if finalize:
            out_ref[cw_rows(c_cw)] = acc_cw * scale
            out_ref[ccw_rows(c_ccw)] = acc_ccw * scale
        if not is_rs:
            out_ref[cw_rows(c_cw)] = (
                cw_ref[slot_r, :, :].astype(jnp.float32) * scale)
            out_ref[ccw_rows(c_ccw)] = (
                ccw_ref[slot_r, :, :].astype(jnp.float32) * scale)


def kernel(x, w_mat, scale_x, scale_w):
    ring = jnp.array(RING, dtype=jnp.int32)
    my = lax.axis_index("i")
    pos = jnp.argmax(ring == my).astype(jnp.int32)
    meta = jnp.stack([pos, ring[(pos - 1) % N_DEV], ring[(pos + 1) % N_DEV]])
    return pl.pallas_call(
        _body,
        out_shape=jax.ShapeDtypeStruct((M, N_OUT), jnp.float32),
        in_specs=[
            pl.BlockSpec(memory_space=pltpu.VMEM),
            pl.BlockSpec(memory_space=pltpu.VMEM),
            pl.BlockSpec(memory_space=pltpu.SMEM),
            pl.BlockSpec(memory_space=pltpu.SMEM),
            pl.BlockSpec(memory_space=pltpu.SMEM),
        ],
        out_specs=pl.BlockSpec(memory_space=pltpu.VMEM),
        scratch_shapes=[
            pltpu.VMEM((M, N_OUT), COMM_DTYPE),
            pltpu.VMEM((2, CHUNK, HALF), COMM_DTYPE),
            pltpu.VMEM((2, CHUNK, HALF), COMM_DTYPE),
            pltpu.SemaphoreType.DMA((2,)),
            pltpu.SemaphoreType.DMA((2,)),
            pltpu.SemaphoreType.DMA((2,)),
            pltpu.SemaphoreType.DMA((2,)),
            pltpu.SemaphoreType.REGULAR,
            pltpu.SemaphoreType.REGULAR,
        ],
        compiler_params=pltpu.CompilerParams(
            collective_id=0,
            vmem_limit_bytes=100 * 1024 * 1024,
        ),
    )(x, w_mat, scale_x, scale_w, meta)


# device time: 221783 ns/iter; 1.6766x vs baseline; 1.3338x over previous
import jax
import jax.numpy as jnp
from jax import lax
from jax.experimental import pallas as pl
from jax.experimental.pallas import tpu as pltpu

N_DEV = 16
M = 4096
N_OUT = 2048
CHUNK = M // N_DEV
N_LANES = 4
LANE_COLS = N_OUT // N_LANES
NSLOT = 3
COMM_DTYPE = jnp.bfloat16

MESH = pl.DeviceIdType.MESH
N_STEPS = 2 * (N_DEV - 1)

RING = (0, 1, 5, 9, 13, 14, 10, 6, 2, 3, 7, 11, 15, 12, 8, 4)

STREAMS = ((+1, 0), (-1, 2 * LANE_COLS), (+1, LANE_COLS), (-1, 3 * LANE_COLS))


def _body(x_ref, w_ref, sx_ref, sw_ref, meta_ref, out_ref, p_ref,
          b0, b1, b2, b3, ss0, rs0, ss1, rs1, ss2, rs2, ss3, rs3,
          cr0, cr1, cr2, cr3):
    pos = meta_ref[0]
    left = meta_ref[1]
    right = meta_ref[2]
    bufs = (b0, b1, b2, b3)
    send_sems = (ss0, ss1, ss2, ss3)
    recv_sems = (rs0, rs1, rs2, rs3)
    credits = (cr0, cr1, cr2, cr3)

    barrier = pltpu.get_barrier_semaphore()
    for nbr in (left, right):
        pl.semaphore_signal(barrier, inc=1, device_id=(nbr,),
                            device_id_type=MESH)
    pl.semaphore_wait(barrier, 2)

    scale = sx_ref[0] * sw_ref[0]
    p_ref[:, :] = lax.dot_general(
        x_ref[:, :].astype(jnp.bfloat16),
        w_ref[:, :].astype(jnp.bfloat16),
        (((1,), (0,)), ((), ())),
        preferred_element_type=jnp.float32,
    ).astype(COMM_DTYPE)

    def rows(c, k):
        return (pl.ds(c * CHUNK, CHUNK), pl.ds(STREAMS[k][1], LANE_COLS))

    def chunk_of(k, g):
        d, _ = STREAMS[k]
        if g < N_DEV - 1:
            return (pos - d * (g + 1)) % N_DEV
        t = g - (N_DEV - 1)
        return (pos - d * t) % N_DEV

    def mk(k, g):
        d, _ = STREAMS[k]
        return pltpu.make_async_remote_copy(
            src_ref=bufs[k].at[g % NSLOT],
            dst_ref=bufs[k].at[(g + 1) % NSLOT],
            send_sem=send_sems[k].at[g % NSLOT],
            recv_sem=recv_sems[k].at[(g + 1) % NSLOT],
            device_id=(right,) if d > 0 else (left,),
            device_id_type=MESH,
        )

    for k in range(N_LANES):
        bufs[k][0, :, :] = p_ref[rows(pos, k)]
        mk(k, 0).start()

    for g in range(N_STEPS):
        slot_r = (g + 1) % NSLOT
        rdmas = [mk(k, g) for k in range(N_LANES)]
        is_rs = g < N_DEV - 1
        finalize = g == N_DEV - 2
        accs = [None] * N_LANES
        for k in range(N_LANES):
            rdmas[k].wait_recv()
            if is_rs:
                acc = (bufs[k][slot_r, :, :].astype(jnp.float32)
                       + p_ref[rows(chunk_of(k, g), k)].astype(jnp.float32))
                bufs[k][slot_r, :, :] = acc.astype(COMM_DTYPE)
                accs[k] = acc
            if g + 1 < N_STEPS:
                if g + 1 >= NSLOT - 1:
                    pl.semaphore_wait(credits[k], 1)
                mk(k, g + 1).start()
        for k in range(N_LANES):
            rdmas[k].wait_send()
            if g <= N_STEPS - NSLOT:
                d, _ = STREAMS[k]
                pl.semaphore_signal(
                    credits[k], inc=1,
                    device_id=(left,) if d > 0 else (right,),
                    device_id_type=MESH)
        if finalize:
            for k in range(N_LANES):
                out_ref[rows(chunk_of(k, g), k)] = accs[k] * scale
        if not is_rs:
            for k in range(N_LANES):
                out_ref[rows(chunk_of(k, g), k)] = (
                    bufs[k][slot_r, :, :].astype(jnp.float32) * scale)


def kernel(x, w_mat, scale_x, scale_w):
    ring = jnp.array(RING, dtype=jnp.int32)
    my = lax.axis_index("i")
    pos = jnp.argmax(ring == my).astype(jnp.int32)
    meta = jnp.stack([pos, ring[(pos - 1) % N_DEV], ring[(pos + 1) % N_DEV]])
    comm = pltpu.VMEM((NSLOT, CHUNK, LANE_COLS), COMM_DTYPE)
    dma = pltpu.SemaphoreType.DMA((NSLOT,))
    return pl.pallas_call(
        _body,
        out_shape=jax.ShapeDtypeStruct((M, N_OUT), jnp.float32),
        in_specs=[
            pl.BlockSpec(memory_space=pltpu.VMEM),
            pl.BlockSpec(memory_space=pltpu.VMEM),
            pl.BlockSpec(memory_space=pltpu.SMEM),
            pl.BlockSpec(memory_space=pltpu.SMEM),
            pl.BlockSpec(memory_space=pltpu.SMEM),
        ],
        out_specs=pl.BlockSpec(memory_space=pltpu.VMEM),
        scratch_shapes=[
            pltpu.VMEM((M, N_OUT), COMM_DTYPE),
            comm, comm, comm, comm,
            dma, dma, dma, dma, dma, dma, dma, dma,
            pltpu.SemaphoreType.REGULAR,
            pltpu.SemaphoreType.REGULAR,
            pltpu.SemaphoreType.REGULAR,
            pltpu.SemaphoreType.REGULAR,
        ],
        compiler_params=pltpu.CompilerParams(
            collective_id=0,
            vmem_limit_bytes=100 * 1024 * 1024,
        ),
    )(x, w_mat, scale_x, scale_w, meta)


# device time: 216955 ns/iter; 1.7139x vs baseline; 1.0223x over previous
import jax
import jax.numpy as jnp
from jax import lax
from jax.experimental import pallas as pl
from jax.experimental.pallas import tpu as pltpu

N_DEV = 16
M = 4096
K_SHARD = 256
N_OUT = 2048
CHUNK = M // N_DEV
N_LANES = 4
LANE_COLS = N_OUT // N_LANES
NSLOT = 3
COMM_DTYPE = jnp.bfloat16

MESH = pl.DeviceIdType.MESH
N_STEPS = 2 * (N_DEV - 1)

RING = (0, 1, 5, 9, 13, 14, 10, 6, 2, 3, 7, 11, 15, 12, 8, 4)

STREAMS = ((+1, 0), (-1, 2 * LANE_COLS), (+1, LANE_COLS), (-1, 3 * LANE_COLS))


def _body(x_ref, w_ref, sx_ref, sw_ref, meta_ref, out_ref, p_ref, w16_ref,
          b0, b1, b2, b3, ss0, rs0, ss1, rs1, ss2, rs2, ss3, rs3,
          cr0, cr1, cr2, cr3):
    pos = meta_ref[0]
    left = meta_ref[1]
    right = meta_ref[2]
    bufs = (b0, b1, b2, b3)
    send_sems = (ss0, ss1, ss2, ss3)
    recv_sems = (rs0, rs1, rs2, rs3)
    credits = (cr0, cr1, cr2, cr3)

    barrier = pltpu.get_barrier_semaphore()
    for nbr in (left, right):
        pl.semaphore_signal(barrier, inc=1, device_id=(nbr,),
                            device_id_type=MESH)
    pl.semaphore_wait(barrier, 2)

    scale = sx_ref[0] * sw_ref[0]
    w16_ref[:, :] = w_ref[:, :].astype(jnp.bfloat16)

    def gemm_chunk(c):
        return lax.dot_general(
            x_ref[pl.ds(c * CHUNK, CHUNK), :].astype(jnp.bfloat16),
            w16_ref[:, :],
            (((1,), (0,)), ((), ())),
            preferred_element_type=jnp.float32,
        ).astype(COMM_DTYPE)

    def rows(c, k):
        return (pl.ds(c * CHUNK, CHUNK), pl.ds(STREAMS[k][1], LANE_COLS))

    def chunk_of(k, g):
        d, _ = STREAMS[k]
        if g < N_DEV - 1:
            return (pos - d * (g + 1)) % N_DEV
        t = g - (N_DEV - 1)
        return (pos - d * t) % N_DEV

    def mk(k, g):
        d, _ = STREAMS[k]
        return pltpu.make_async_remote_copy(
            src_ref=bufs[k].at[g % NSLOT],
            dst_ref=bufs[k].at[(g + 1) % NSLOT],
            send_sem=send_sems[k].at[g % NSLOT],
            recv_sem=recv_sems[k].at[(g + 1) % NSLOT],
            device_id=(right,) if d > 0 else (left,),
            device_id_type=MESH,
        )

    seed = gemm_chunk(pos)
    for k in range(N_LANES):
        c0 = STREAMS[k][1]
        bufs[k][0, :, :] = seed[:, c0:c0 + LANE_COLS]
        mk(k, 0).start()

    for g in range(N_STEPS):
        slot_r = (g + 1) % NSLOT
        rdmas = [mk(k, g) for k in range(N_LANES)]
        is_rs = g < N_DEV - 1
        finalize = g == N_DEV - 2
        if g <= 7:
            p_ref[(pl.ds(((pos - g - 1) % N_DEV) * CHUNK, CHUNK),
                   slice(None))] = gemm_chunk((pos - g - 1) % N_DEV)
        if g <= 6:
            p_ref[(pl.ds(((pos + g + 1) % N_DEV) * CHUNK, CHUNK),
                   slice(None))] = gemm_chunk((pos + g + 1) % N_DEV)
        accs = [None] * N_LANES
        for k in range(N_LANES):
            rdmas[k].wait_recv()
            if is_rs:
                acc = (bufs[k][slot_r, :, :].astype(jnp.float32)
                       + p_ref[rows(chunk_of(k, g), k)].astype(jnp.float32))
                bufs[k][slot_r, :, :] = acc.astype(COMM_DTYPE)
                accs[k] = acc
            if g + 1 < N_STEPS:
                if g + 1 >= NSLOT - 1:
                    pl.semaphore_wait(credits[k], 1)
                mk(k, g + 1).start()
        for k in range(N_LANES):
            rdmas[k].wait_send()
            if g <= N_STEPS - NSLOT:
                d, _ = STREAMS[k]
                pl.semaphore_signal(
                    credits[k], inc=1,
                    device_id=(left,) if d > 0 else (right,),
                    device_id_type=MESH)
        if finalize:
            for k in range(N_LANES):
                out_ref[rows(chunk_of(k, g), k)] = accs[k] * scale
        if not is_rs:
            for k in range(N_LANES):
                out_ref[rows(chunk_of(k, g), k)] = (
                    bufs[k][slot_r, :, :].astype(jnp.float32) * scale)


def kernel(x, w_mat, scale_x, scale_w):
    ring = jnp.array(RING, dtype=jnp.int32)
    my = lax.axis_index("i")
    pos = jnp.argmax(ring == my).astype(jnp.int32)
    meta = jnp.stack([pos, ring[(pos - 1) % N_DEV], ring[(pos + 1) % N_DEV]])
    comm = pltpu.VMEM((NSLOT, CHUNK, LANE_COLS), COMM_DTYPE)
    dma = pltpu.SemaphoreType.DMA((NSLOT,))
    return pl.pallas_call(
        _body,
        out_shape=jax.ShapeDtypeStruct((M, N_OUT), jnp.float32),
        in_specs=[
            pl.BlockSpec(memory_space=pltpu.VMEM),
            pl.BlockSpec(memory_space=pltpu.VMEM),
            pl.BlockSpec(memory_space=pltpu.SMEM),
            pl.BlockSpec(memory_space=pltpu.SMEM),
            pl.BlockSpec(memory_space=pltpu.SMEM),
        ],
        out_specs=pl.BlockSpec(memory_space=pltpu.VMEM),
        scratch_shapes=[
            pltpu.VMEM((M, N_OUT), COMM_DTYPE),
            pltpu.VMEM((K_SHARD, N_OUT), COMM_DTYPE),
            comm, comm, comm, comm,
            dma, dma, dma, dma, dma, dma, dma, dma,
            pltpu.SemaphoreType.REGULAR,
            pltpu.SemaphoreType.REGULAR,
            pltpu.SemaphoreType.REGULAR,
            pltpu.SemaphoreType.REGULAR,
        ],
        compiler_params=pltpu.CompilerParams(
            collective_id=0,
            vmem_limit_bytes=100 * 1024 * 1024,
        ),
    )(x, w_mat, scale_x, scale_w, meta)
